# baseline (speedup 1.0000x reference)
import functools

import jax
import jax.numpy as jnp
from jax import lax
from jax.experimental import pallas as pl
from jax.experimental.pallas import tpu as pltpu
from jax.experimental.pallas import tpu_sc as plsc

_L = 16
_ROWS = 2
_D = 64


def _sc_body(x_hbm, out_hbm, rows_v, res_v):
    cid = lax.axis_index("c")
    sid = lax.axis_index("s")
    wid = sid * 2 + cid

    @pl.when(wid == 0)
    def _():
        pltpu.sync_copy(x_hbm, rows_v)
        lane = lax.iota(jnp.int32, _L)
        acc = jnp.zeros((_L,), jnp.int32)
        for row, skip_col in ((0, 2), (1, -1)):
            for chunk in range(_D // _L):
                v = rows_v[row, pl.ds(chunk * _L, _L)]
                neq = jnp.where(v != jnp.float32(1.0),
                                jnp.int32(1), jnp.int32(0))
                if 0 <= skip_col - chunk * _L < _L:
                    neq = jnp.where(lane != jnp.int32(skip_col - chunk * _L),
                                    neq, jnp.int32(0))
                acc = acc | neq
        res_v[...] = acc | jnp.int32(1)
        pltpu.sync_copy(res_v, out_hbm)


_sc_diff = functools.partial(
    pl.kernel,
    mesh=plsc.VectorSubcoreMesh(core_axis_name="c", subcore_axis_name="s",
                                num_cores=1),
    out_type=jax.ShapeDtypeStruct((_L,), jnp.int32),
    scratch_types=[
        pltpu.VMEM((_ROWS, _D), jnp.float32),
        pltpu.VMEM((_L,), jnp.int32),
    ],
)(_sc_body)


def kernel(x):
    out = _sc_diff(lax.slice(x, (1, 0), (1 + _ROWS, _D)))
    return out[0].astype(jnp.bool_)
